# baseline (speedup 1.0000x reference)
import functools

import jax
import jax.numpy as jnp
from jax.experimental import pallas as pl
from jax.experimental.pallas import tpu as pltpu


def _body(x_ref, nx_ref, wl_ref, wr_ref, b_ref, o_ref, *, k):
    mean = jnp.sum(nx_ref[...], axis=1) * (1.0 / k)
    o_ref[...] = (
        jnp.dot(x_ref[...], wl_ref[...], preferred_element_type=jnp.float32)
        + jnp.dot(mean, wr_ref[...], preferred_element_type=jnp.float32)
        + b_ref[...]
    )


def kernel(x, neigh_x, W_l, b_l, W_r, b_r):
    n, k, d_in = neigh_x.shape
    d_out = W_l.shape[0]
    block = 512
    grid = (pl.cdiv(n, block),)

    wl_t = W_l.T
    wr_t = W_r.T
    bias = (b_l + b_r).reshape(1, d_out)

    out = pl.pallas_call(
        functools.partial(_body, k=k),
        grid=grid,
        in_specs=[
            pl.BlockSpec((block, d_in), lambda i: (i, 0)),
            pl.BlockSpec((block, k, d_in), lambda i: (i, 0, 0)),
            pl.BlockSpec((d_in, d_out), lambda i: (0, 0)),
            pl.BlockSpec((d_in, d_out), lambda i: (0, 0)),
            pl.BlockSpec((1, d_out), lambda i: (0, 0)),
        ],
        out_specs=pl.BlockSpec((block, d_out), lambda i: (i, 0)),
        out_shape=jax.ShapeDtypeStruct((n, d_out), jnp.float32),
        compiler_params=pltpu.CompilerParams(
            dimension_semantics=("arbitrary",),
        ),
    )(x, neigh_x, wl_t, wr_t, bias)
    return out


# speedup vs baseline: 1.0050x; 1.0050x over previous
import functools

import jax
import jax.numpy as jnp
from jax.experimental import pallas as pl
from jax.experimental.pallas import tpu as pltpu


def _body(x_ref, nx_ref, wl_ref, wr_ref, b_ref, o_ref, *, k):
    mean = jnp.sum(nx_ref[...], axis=1) * (1.0 / k)
    o_ref[...] = (
        jnp.dot(x_ref[...], wl_ref[...], preferred_element_type=jnp.float32)
        + jnp.dot(mean, wr_ref[...], preferred_element_type=jnp.float32)
        + b_ref[...]
    )


def kernel(x, neigh_x, W_l, b_l, W_r, b_r):
    n, k, d_in = neigh_x.shape
    d_out = W_l.shape[0]
    block = 400
    grid = (pl.cdiv(n, block),)

    wl_t = W_l.T
    wr_t = W_r.T
    bias = (b_l + b_r).reshape(1, d_out)

    out = pl.pallas_call(
        functools.partial(_body, k=k),
        grid=grid,
        in_specs=[
            pl.BlockSpec((block, d_in), lambda i: (i, 0)),
            pl.BlockSpec((block, k, d_in), lambda i: (i, 0, 0)),
            pl.BlockSpec((d_in, d_out), lambda i: (0, 0)),
            pl.BlockSpec((d_in, d_out), lambda i: (0, 0)),
            pl.BlockSpec((1, d_out), lambda i: (0, 0)),
        ],
        out_specs=pl.BlockSpec((block, d_out), lambda i: (i, 0)),
        out_shape=jax.ShapeDtypeStruct((n, d_out), jnp.float32),
        compiler_params=pltpu.CompilerParams(
            dimension_semantics=("parallel",),
        ),
    )(x, neigh_x, wl_t, wr_t, bias)
    return out
